# baseline (speedup 1.0000x reference)
import functools

import jax
import jax.numpy as jnp
from jax import lax
from jax.experimental import pallas as pl
from jax.experimental.pallas import tpu as pltpu
from jax.experimental.pallas import tpu_sc as plsc

N_NODES = 10000
N_PAD = 10240
N_EDGES = 160000
IN_DIM = 256
OUT_DIM = 256
HALF = 128

NC = 2
NS = 16

BLK = 125
H_NBLK = 40
ABLK = 50
A_NBLK = 200
NSLOT = 4

_mesh = plsc.VectorSubcoreMesh(core_axis_name="c", subcore_axis_name="s")


@functools.partial(
    pl.kernel,
    mesh=_mesh,
    out_type=jax.ShapeDtypeStruct((NC, N_PAD, 16), jnp.float32),
    scratch_types=[
        pltpu.VMEM((H_NBLK, BLK), jnp.int32),
        pltpu.VMEM((BLK, 16), jnp.float32),
        pltpu.VMEM((160, 16), jnp.float32),
        pltpu.VMEM_SHARED((N_PAD, 16), jnp.float32),
        pltpu.SemaphoreType.DMA,
    ],
)
def _sc_hist(hs_hbm, d16_hbm, idx_v, ones_v, zer_v, d_sh, sem):
    c = lax.axis_index("c")
    s = lax.axis_index("s")
    wid = c * NS + s

    @pl.loop(0, BLK)
    def _(j):
        ones_v[j, :] = jnp.full((16,), 1.0, jnp.float32)

    @pl.loop(0, 160)
    def _(j):
        zer_v[j, :] = jnp.zeros((16,), jnp.float32)

    @pl.loop(0, 4)
    def _(k):
        pltpu.sync_copy(zer_v, d_sh.at[pl.ds(s * 640 + k * 160, 160)])

    plsc.subcore_barrier()

    pltpu.sync_copy(hs_hbm.at[pl.ds(wid * H_NBLK, H_NBLK)], idx_v)

    @pl.loop(0, H_NBLK)
    def _(j):
        pltpu.async_copy(ones_v, d_sh.at[idx_v.at[j]], sem, add=True)

    @pl.loop(0, H_NBLK)
    def _(j):
        pltpu.make_async_copy(ones_v, d_sh.at[idx_v.at[j]], sem).wait()

    plsc.subcore_barrier()
    pltpu.sync_copy(d_sh.at[pl.ds(s * 640, 640)],
                    d16_hbm.at[c, pl.ds(s * 640, 640)])


@functools.partial(
    pl.kernel,
    mesh=_mesh,
    out_type=jax.ShapeDtypeStruct((NC, N_PAD, HALF), jnp.float32),
    scratch_types=[
        pltpu.VMEM((A_NBLK // 5, ABLK), jnp.int32),
        pltpu.VMEM((A_NBLK // 5, ABLK), jnp.int32),
        pltpu.VMEM((ABLK, HALF), jnp.float32),
        pltpu.VMEM((ABLK, HALF), jnp.float32),
        pltpu.VMEM((ABLK, HALF), jnp.float32),
        pltpu.VMEM((ABLK, HALF), jnp.float32),
        pltpu.VMEM_SHARED((N_PAD, HALF), jnp.float32),
        pltpu.SemaphoreType.DMA,
        pltpu.SemaphoreType.DMA,
        pltpu.SemaphoreType.DMA,
        pltpu.SemaphoreType.DMA,
        pltpu.SemaphoreType.DMA,
        pltpu.SemaphoreType.DMA,
        pltpu.SemaphoreType.DMA,
        pltpu.SemaphoreType.DMA,
    ],
)
def _sc_agg(scaled_hbm, tsg_hbm, hs_hbm, agg_hbm,
            tsg_v, hs_v, rb0, rb1, rb2, rb3, agg_sh,
            gs0, gs1, gs2, gs3, ss0, ss1, ss2, ss3):
    c = lax.axis_index("c")
    s = lax.axis_index("s")
    ph_nblk = A_NBLK // 5
    rbs = (rb0, rb1, rb2, rb3)
    gss = (gs0, gs1, gs2, gs3)
    sss = (ss0, ss1, ss2, ss3)

    @pl.loop(0, ABLK)
    def _(j):
        @pl.loop(0, HALF // 16)
        def _(k):
            rb0[j, pl.ds(k * 16, 16)] = jnp.zeros((16,), jnp.float32)

    @pl.loop(0, 16)
    def _(k):
        pltpu.sync_copy(rb0.at[pl.ds(0, 40)],
                        agg_sh.at[pl.ds(s * 640 + k * 40, 40)])

    plsc.subcore_barrier()

    scaled_c = scaled_hbm.at[pl.ds(c * N_NODES, N_NODES)]

    def start_gather(b, rb, sem):
        pltpu.make_async_copy(scaled_c.at[tsg_v.at[b]], rb, sem).start()

    def wait_gather(b, rb, sem):
        pltpu.make_async_copy(scaled_c.at[tsg_v.at[b]], rb, sem).wait()

    def start_scatter(b, rb, sem):
        pltpu.async_copy(rb, agg_sh.at[hs_v.at[b]], sem, add=True)

    def wait_scatter(b, rb, sem):
        pltpu.make_async_copy(rb, agg_sh.at[hs_v.at[b]], sem).wait()

    for ph in range(5):
        pltpu.sync_copy(
            tsg_hbm.at[pl.ds(s * A_NBLK + ph * ph_nblk, ph_nblk)], tsg_v)
        pltpu.sync_copy(
            hs_hbm.at[pl.ds(s * A_NBLK + ph * ph_nblk, ph_nblk)], hs_v)

        for p in range(NSLOT):
            start_gather(p, rbs[p], gss[p])

        @pl.loop(0, ph_nblk // NSLOT)
        def _(j):
            b0 = NSLOT * j
            for p in range(NSLOT):
                wait_gather(b0 + p, rbs[p], gss[p])
                start_scatter(b0 + p, rbs[p], sss[p])
            for p in range(NSLOT):
                b = b0 + p

                @pl.when(b + NSLOT < ph_nblk)
                def _():
                    wait_scatter(b, rbs[p], sss[p])
                    start_gather(b + NSLOT, rbs[p], gss[p])

        for p in range(NSLOT):
            wait_scatter(ph_nblk - NSLOT + p, rbs[p], sss[p])

    plsc.subcore_barrier()
    pltpu.sync_copy(agg_sh.at[pl.ds(s * 640, 640)],
                    agg_hbm.at[c, pl.ds(s * 640, 640)])


def _tc_scale_mm(d16, feats, W):
    B = 1000

    def body(d_ref, x_ref, w_ref, o_ref):
        deg = d_ref[0, :, 0:1] + d_ref[1, :, 0:1]
        xs = lax.rsqrt(deg) * x_ref[...]
        y = lax.dot_general(xs, w_ref[...], (((1,), (1,)), ((), ())),
                            preferred_element_type=jnp.float32)
        o_ref[0] = y[:, :HALF]
        o_ref[1] = y[:, HALF:]

    return pl.pallas_call(
        body,
        grid=(N_NODES // B,),
        in_specs=[
            pl.BlockSpec((2, B, 16), lambda i: (0, i, 0)),
            pl.BlockSpec((B, IN_DIM), lambda i: (i, 0)),
            pl.BlockSpec((OUT_DIM, IN_DIM), lambda i: (0, 0)),
        ],
        out_specs=pl.BlockSpec((2, B, HALF), lambda i: (0, i, 0)),
        out_shape=jax.ShapeDtypeStruct((2, N_NODES, HALF), jnp.float32),
    )(d16, feats, W)


def _tc_out(d16, aggp):
    B = 1000

    def body(d_ref, a_ref, o_ref):
        deg = d_ref[0, :, 0:1] + d_ref[1, :, 0:1]
        sc = lax.rsqrt(deg)
        o_ref[:, :HALF] = jnp.maximum(a_ref[0] * sc, 0.0)
        o_ref[:, HALF:] = jnp.maximum(a_ref[1] * sc, 0.0)

    return pl.pallas_call(
        body,
        grid=(N_NODES // B,),
        in_specs=[
            pl.BlockSpec((2, B, 16), lambda i: (0, i, 0)),
            pl.BlockSpec((2, B, HALF), lambda i: (0, i, 0)),
        ],
        out_specs=pl.BlockSpec((B, OUT_DIM), lambda i: (i, 0)),
        out_shape=jax.ShapeDtypeStruct((N_NODES, OUT_DIM), jnp.float32),
    )(d16, aggp)


def kernel(feats_n, edges, W):
    e125 = edges.astype(jnp.int32).reshape(2, N_EDGES // BLK, BLK)
    e50 = edges.astype(jnp.int32).reshape(2, N_EDGES // ABLK, ABLK)
    hsp = e125[0]
    hs_a = e50[0]
    tsg = e50[1]

    d16 = _sc_hist(hsp)
    scaled = _tc_scale_mm(d16, feats_n, W)
    scaled2 = scaled.reshape(2 * N_NODES, HALF)
    aggp = _sc_agg(scaled2, tsg, hs_a)
    return _tc_out(d16, aggp)


# speedup vs baseline: 1.1125x; 1.1125x over previous
import functools

import jax
import jax.numpy as jnp
from jax import lax
from jax.experimental import pallas as pl
from jax.experimental.pallas import tpu as pltpu
from jax.experimental.pallas import tpu_sc as plsc

N_NODES = 10000
N_PAD = 10240
N_EDGES = 160000
IN_DIM = 256
OUT_DIM = 256
HALF = 128

NC = 2
NS = 16

BLK = 125
H_NBLK = 40
A_NBLK = 80

_mesh = plsc.VectorSubcoreMesh(core_axis_name="c", subcore_axis_name="s")


@functools.partial(
    pl.kernel,
    mesh=_mesh,
    out_type=jax.ShapeDtypeStruct((NC, N_PAD, 16), jnp.float32),
    scratch_types=[
        pltpu.VMEM((H_NBLK, BLK), jnp.int32),
        pltpu.VMEM((BLK, 16), jnp.float32),
        pltpu.VMEM((160, 16), jnp.float32),
        pltpu.VMEM_SHARED((N_PAD, 16), jnp.float32),
        pltpu.SemaphoreType.DMA,
    ],
)
def _sc_hist(hs_hbm, d16_hbm, idx_v, ones_v, zer_v, d_sh, sem):
    c = lax.axis_index("c")
    s = lax.axis_index("s")
    wid = c * NS + s

    @pl.loop(0, BLK)
    def _(j):
        ones_v[j, :] = jnp.full((16,), 1.0, jnp.float32)

    @pl.loop(0, 160)
    def _(j):
        zer_v[j, :] = jnp.zeros((16,), jnp.float32)

    @pl.loop(0, 4)
    def _(k):
        pltpu.sync_copy(zer_v, d_sh.at[pl.ds(s * 640 + k * 160, 160)])

    plsc.subcore_barrier()

    pltpu.sync_copy(hs_hbm.at[pl.ds(wid * H_NBLK, H_NBLK)], idx_v)

    @pl.loop(0, H_NBLK)
    def _(j):
        pltpu.async_copy(ones_v, d_sh.at[idx_v.at[j]], sem, add=True)

    @pl.loop(0, H_NBLK)
    def _(j):
        pltpu.make_async_copy(ones_v, d_sh.at[idx_v.at[j]], sem).wait()

    plsc.subcore_barrier()
    pltpu.sync_copy(d_sh.at[pl.ds(s * 640, 640)],
                    d16_hbm.at[c, pl.ds(s * 640, 640)])


@functools.partial(
    pl.kernel,
    mesh=_mesh,
    out_type=jax.ShapeDtypeStruct((NC, N_PAD, HALF), jnp.float32),
    scratch_types=[
        pltpu.VMEM((A_NBLK // 2, BLK), jnp.int32),
        pltpu.VMEM((A_NBLK // 2, BLK), jnp.int32),
        pltpu.VMEM((BLK, HALF), jnp.float32),
        pltpu.VMEM((BLK, HALF), jnp.float32),
        pltpu.VMEM_SHARED((N_PAD, HALF), jnp.float32),
        pltpu.SemaphoreType.DMA,
        pltpu.SemaphoreType.DMA,
    ],
)
def _sc_agg(scaled_hbm, tsg_hbm, hs_hbm, agg_hbm,
            tsg_v, hs_v, rb0, rb1, agg_sh, sem0, sem1):
    c = lax.axis_index("c")
    s = lax.axis_index("s")
    half_nblk = A_NBLK // 2

    @pl.loop(0, BLK)
    def _(j):
        @pl.loop(0, HALF // 16)
        def _(k):
            rb0[j, pl.ds(k * 16, 16)] = jnp.zeros((16,), jnp.float32)

    @pl.loop(0, 8)
    def _(k):
        pltpu.sync_copy(rb0.at[pl.ds(0, 80)],
                        agg_sh.at[pl.ds(s * 640 + k * 80, 80)])

    plsc.subcore_barrier()

    scaled_c = scaled_hbm.at[pl.ds(c * N_NODES, N_NODES)]

    def start_gather(b, rb, sem):
        pltpu.make_async_copy(scaled_c.at[tsg_v.at[b]], rb, sem).start()

    def wait_gather(b, rb, sem):
        pltpu.make_async_copy(scaled_c.at[tsg_v.at[b]], rb, sem).wait()

    def scatter(b, rb):
        pltpu.sync_copy(rb, agg_sh.at[hs_v.at[b]], add=True)

    for ph in range(2):
        pltpu.sync_copy(
            tsg_hbm.at[pl.ds(s * A_NBLK + ph * half_nblk, half_nblk)], tsg_v)
        pltpu.sync_copy(
            hs_hbm.at[pl.ds(s * A_NBLK + ph * half_nblk, half_nblk)], hs_v)

        start_gather(0, rb0, sem0)
        start_gather(1, rb1, sem1)

        @pl.loop(0, half_nblk // 2)
        def _(j):
            b0 = 2 * j

            def step(b, rb, sem):
                wait_gather(b, rb, sem)
                scatter(b, rb)

                @pl.when(b + 2 < half_nblk)
                def _():
                    start_gather(b + 2, rb, sem)

            step(b0, rb0, sem0)
            step(b0 + 1, rb1, sem1)

    plsc.subcore_barrier()
    pltpu.sync_copy(agg_sh.at[pl.ds(s * 640, 640)],
                    agg_hbm.at[c, pl.ds(s * 640, 640)])


def _tc_mm(feats, W):
    B = 2000

    def body(x_ref, w_ref, o_ref):
        o_ref[...] = lax.dot_general(x_ref[...], w_ref[...],
                                     (((1,), (1,)), ((), ())),
                                     preferred_element_type=jnp.float32)

    return pl.pallas_call(
        body,
        grid=(N_NODES // B,),
        in_specs=[
            pl.BlockSpec((B, IN_DIM), lambda i: (i, 0)),
            pl.BlockSpec((OUT_DIM, IN_DIM), lambda i: (0, 0)),
        ],
        out_specs=pl.BlockSpec((B, OUT_DIM), lambda i: (i, 0)),
        out_shape=jax.ShapeDtypeStruct((N_NODES, OUT_DIM), jnp.float32),
    )(feats, W)


def _tc_scale(d16, y):
    B = 2000

    def body(d_ref, x_ref, o_ref):
        deg = d_ref[0, :, 0:1] + d_ref[1, :, 0:1]
        xs = lax.rsqrt(deg) * x_ref[...]
        o_ref[0] = xs[:, :HALF]
        o_ref[1] = xs[:, HALF:]

    return pl.pallas_call(
        body,
        grid=(N_NODES // B,),
        in_specs=[
            pl.BlockSpec((2, B, 16), lambda i: (0, i, 0)),
            pl.BlockSpec((B, IN_DIM), lambda i: (i, 0)),
        ],
        out_specs=pl.BlockSpec((2, B, HALF), lambda i: (0, i, 0)),
        out_shape=jax.ShapeDtypeStruct((2, N_NODES, HALF), jnp.float32),
    )(d16, y)


def _tc_out(d16, aggp):
    B = 2000

    def body(d_ref, a_ref, o_ref):
        deg = d_ref[0, :, 0:1] + d_ref[1, :, 0:1]
        sc = lax.rsqrt(deg)
        o_ref[:, :HALF] = jnp.maximum(a_ref[0] * sc, 0.0)
        o_ref[:, HALF:] = jnp.maximum(a_ref[1] * sc, 0.0)

    return pl.pallas_call(
        body,
        grid=(N_NODES // B,),
        in_specs=[
            pl.BlockSpec((2, B, 16), lambda i: (0, i, 0)),
            pl.BlockSpec((2, B, HALF), lambda i: (0, i, 0)),
        ],
        out_specs=pl.BlockSpec((B, OUT_DIM), lambda i: (i, 0)),
        out_shape=jax.ShapeDtypeStruct((N_NODES, OUT_DIM), jnp.float32),
    )(d16, aggp)


def kernel(feats_n, edges, W):
    e3 = edges.astype(jnp.int32).reshape(2, N_EDGES // BLK, BLK)
    hsp = e3[0]
    tsg = e3[1]

    y = _tc_mm(feats_n, W)
    d16 = _sc_hist(hsp)
    scaled = _tc_scale(d16, y)
    scaled2 = scaled.reshape(2 * N_NODES, HALF)
    aggp = _sc_agg(scaled2, tsg, hsp)
    return _tc_out(d16, aggp)


# speedup vs baseline: 1.1337x; 1.0190x over previous
import functools

import jax
import jax.numpy as jnp
from jax import lax
from jax.experimental import pallas as pl
from jax.experimental.pallas import tpu as pltpu
from jax.experimental.pallas import tpu_sc as plsc

N_NODES = 10000
N_PAD = 10240
N_EDGES = 160000
IN_DIM = 256
OUT_DIM = 256
HALF = 128

NC = 2
NS = 16

BLK = 125
H_NBLK = 40
A_NBLK = 80

_mesh = plsc.VectorSubcoreMesh(core_axis_name="c", subcore_axis_name="s")


@functools.partial(
    pl.kernel,
    mesh=_mesh,
    out_type=jax.ShapeDtypeStruct((NC, N_PAD, 16), jnp.float32),
    scratch_types=[
        pltpu.VMEM((H_NBLK, BLK), jnp.int32),
        pltpu.VMEM((BLK, 16), jnp.float32),
        pltpu.VMEM((160, 16), jnp.float32),
        pltpu.VMEM_SHARED((N_PAD, 16), jnp.float32),
        pltpu.SemaphoreType.DMA,
    ],
)
def _sc_hist(e3_hbm, d16_hbm, idx_v, ones_v, zer_v, d_sh, sem):
    c = lax.axis_index("c")
    s = lax.axis_index("s")
    wid = c * NS + s
    hs_hbm = e3_hbm.at[0]

    @pl.loop(0, BLK)
    def _(j):
        ones_v[j, :] = jnp.full((16,), 1.0, jnp.float32)

    @pl.loop(0, 160)
    def _(j):
        zer_v[j, :] = jnp.zeros((16,), jnp.float32)

    @pl.loop(0, 4)
    def _(k):
        pltpu.sync_copy(zer_v, d_sh.at[pl.ds(s * 640 + k * 160, 160)])

    plsc.subcore_barrier()

    pltpu.sync_copy(hs_hbm.at[pl.ds(wid * H_NBLK, H_NBLK)], idx_v)

    @pl.loop(0, H_NBLK)
    def _(j):
        pltpu.async_copy(ones_v, d_sh.at[idx_v.at[j]], sem, add=True)

    @pl.loop(0, H_NBLK)
    def _(j):
        pltpu.make_async_copy(ones_v, d_sh.at[idx_v.at[j]], sem).wait()

    plsc.subcore_barrier()
    pltpu.sync_copy(d_sh.at[pl.ds(s * 640, 640)],
                    d16_hbm.at[c, pl.ds(s * 640, 640)])


@functools.partial(
    pl.kernel,
    mesh=_mesh,
    out_type=jax.ShapeDtypeStruct((NC, N_PAD, HALF), jnp.float32),
    scratch_types=[
        pltpu.VMEM((A_NBLK // 2, BLK), jnp.int32),
        pltpu.VMEM((A_NBLK // 2, BLK), jnp.int32),
        pltpu.VMEM((BLK, HALF), jnp.float32),
        pltpu.VMEM((BLK, HALF), jnp.float32),
        pltpu.VMEM_SHARED((N_PAD, HALF), jnp.float32),
        pltpu.SemaphoreType.DMA,
        pltpu.SemaphoreType.DMA,
    ],
)
def _sc_agg(scaled_hbm, e3_hbm, agg_hbm,
            tsg_v, hs_v, rb0, rb1, agg_sh, sem0, sem1):
    c = lax.axis_index("c")
    s = lax.axis_index("s")
    half_nblk = A_NBLK // 2
    hs_hbm = e3_hbm.at[0]
    tsg_hbm = e3_hbm.at[1]

    @pl.loop(0, BLK)
    def _(j):
        @pl.loop(0, HALF // 16)
        def _(k):
            rb0[j, pl.ds(k * 16, 16)] = jnp.zeros((16,), jnp.float32)

    @pl.loop(0, 8)
    def _(k):
        pltpu.sync_copy(rb0.at[pl.ds(0, 80)],
                        agg_sh.at[pl.ds(s * 640 + k * 80, 80)])

    plsc.subcore_barrier()

    scaled_c = scaled_hbm.at[pl.ds(c * N_NODES, N_NODES)]

    def start_gather(b, rb, sem):
        pltpu.make_async_copy(scaled_c.at[tsg_v.at[b]], rb, sem).start()

    def wait_gather(b, rb, sem):
        pltpu.make_async_copy(scaled_c.at[tsg_v.at[b]], rb, sem).wait()

    def scatter(b, rb):
        pltpu.sync_copy(rb, agg_sh.at[hs_v.at[b]], add=True)

    for ph in range(2):
        pltpu.sync_copy(
            tsg_hbm.at[pl.ds(s * A_NBLK + ph * half_nblk, half_nblk)], tsg_v)
        pltpu.sync_copy(
            hs_hbm.at[pl.ds(s * A_NBLK + ph * half_nblk, half_nblk)], hs_v)

        start_gather(0, rb0, sem0)
        start_gather(1, rb1, sem1)

        @pl.loop(0, half_nblk // 2)
        def _(j):
            b0 = 2 * j

            def step(b, rb, sem):
                wait_gather(b, rb, sem)
                scatter(b, rb)

                @pl.when(b + 2 < half_nblk)
                def _():
                    start_gather(b + 2, rb, sem)

            step(b0, rb0, sem0)
            step(b0 + 1, rb1, sem1)

    plsc.subcore_barrier()
    pltpu.sync_copy(agg_sh.at[pl.ds(s * 640, 640)],
                    agg_hbm.at[c, pl.ds(s * 640, 640)])


def _tc_mm(feats, W):
    B = 2000

    def body(x_ref, w_ref, o_ref):
        o_ref[...] = lax.dot_general(x_ref[...], w_ref[...],
                                     (((1,), (1,)), ((), ())),
                                     preferred_element_type=jnp.float32)

    return pl.pallas_call(
        body,
        grid=(N_NODES // B,),
        in_specs=[
            pl.BlockSpec((B, IN_DIM), lambda i: (i, 0)),
            pl.BlockSpec((OUT_DIM, IN_DIM), lambda i: (0, 0)),
        ],
        out_specs=pl.BlockSpec((B, OUT_DIM), lambda i: (i, 0)),
        out_shape=jax.ShapeDtypeStruct((N_NODES, OUT_DIM), jnp.float32),
    )(feats, W)


def _tc_scale(d16, y):
    B = 2000

    def body(d_ref, x_ref, o_ref):
        deg = d_ref[0, :, 0:1] + d_ref[1, :, 0:1]
        xs = lax.rsqrt(deg) * x_ref[...]
        o_ref[0] = xs[:, :HALF]
        o_ref[1] = xs[:, HALF:]

    return pl.pallas_call(
        body,
        grid=(N_NODES // B,),
        in_specs=[
            pl.BlockSpec((2, B, 16), lambda i: (0, i, 0)),
            pl.BlockSpec((B, IN_DIM), lambda i: (i, 0)),
        ],
        out_specs=pl.BlockSpec((2, B, HALF), lambda i: (0, i, 0)),
        out_shape=jax.ShapeDtypeStruct((2, N_NODES, HALF), jnp.float32),
    )(d16, y)


def _tc_out(d16, aggp):
    B = 2000

    def body(d_ref, a_ref, o_ref):
        deg = d_ref[0, :, 0:1] + d_ref[1, :, 0:1]
        sc = lax.rsqrt(deg)
        o_ref[:, :HALF] = jnp.maximum(a_ref[0] * sc, 0.0)
        o_ref[:, HALF:] = jnp.maximum(a_ref[1] * sc, 0.0)

    return pl.pallas_call(
        body,
        grid=(N_NODES // B,),
        in_specs=[
            pl.BlockSpec((2, B, 16), lambda i: (0, i, 0)),
            pl.BlockSpec((2, B, HALF), lambda i: (0, i, 0)),
        ],
        out_specs=pl.BlockSpec((B, OUT_DIM), lambda i: (i, 0)),
        out_shape=jax.ShapeDtypeStruct((N_NODES, OUT_DIM), jnp.float32),
    )(d16, aggp)


def kernel(feats_n, edges, W):
    e3 = edges.astype(jnp.int32).reshape(2, N_EDGES // BLK, BLK)

    y = _tc_mm(feats_n, W)
    d16 = _sc_hist(e3)
    scaled = _tc_scale(d16, y)
    scaled2 = scaled.reshape(2 * N_NODES, HALF)
    aggp = _sc_agg(scaled2, e3)
    return _tc_out(d16, aggp)
